# baseline (speedup 1.0000x reference)
import functools
import jax
import jax.numpy as jnp
from jax import lax
from jax.experimental import pallas as pl
from jax.experimental.pallas import tpu as pltpu
from jax.experimental.pallas import tpu_sc as plsc

B, N, D = 4, 2048, 64
C = 2 * D
KN = 32
BN = B * N
CNT = BN * KN
IDXW = 40
GL = KN + 1
F32 = jnp.float32
I32 = jnp.int32

_PREP_ROWS = 512


def _prep_body(x_ref, w1_ref, a_ref, bm_ref):
    x = x_ref[...]
    w1a = w1_ref[:, :D]
    w1d = w1_ref[:, D:] - w1a
    dn = (((1,), (1,)), ((), ()))
    a_ref[...] = lax.dot_general(x, w1a, dn)
    bm_ref[...] = lax.dot_general(x, w1d, dn)


def _prep(x_flat, W1):
    return pl.pallas_call(
        _prep_body,
        grid=(BN // _PREP_ROWS,),
        in_specs=[
            pl.BlockSpec((_PREP_ROWS, D), lambda i: (i, 0)),
            pl.BlockSpec((C, C), lambda i: (0, 0)),
        ],
        out_specs=[
            pl.BlockSpec((_PREP_ROWS, C), lambda i: (i, 0)),
            pl.BlockSpec((_PREP_ROWS, C), lambda i: (i, 0)),
        ],
        out_shape=[
            jax.ShapeDtypeStruct((BN, C), F32),
            jax.ShapeDtypeStruct((BN, C), F32),
        ],
    )(x_flat, W1)


_BQ = 1024


def _knn_body(b, cq_ref, cf_ref, idx_ref):
    i = pl.program_id(0)
    cq = cq_ref[0]
    cf = cf_ref[0]
    sqq = jnp.sum(cq * cq, axis=0)[:, None]
    sqf = jnp.sum(cf * cf, axis=0)[None, :]
    dot = lax.dot_general(cq, cf, (((0,), (0,)), ((), ())))
    d2 = sqq + sqf - 2.0 * dot
    iota = lax.broadcasted_iota(I32, (_BQ, N), 1)
    ik = lax.broadcasted_iota(I32, (_BQ, IDXW), 1)
    inf = jnp.float32(jnp.inf)

    row = lax.broadcasted_iota(I32, (_BQ, IDXW), 0) + (i * _BQ + (B + b) * N)
    out = jnp.where(ik == 0, row, 0)
    m = jnp.min(d2, axis=1, keepdims=True)
    for u in range(KN):
        cand = jnp.where(d2 == m, iota, N)
        am = jnp.min(cand, axis=1, keepdims=True)
        d2 = jnp.where(iota == am, inf, d2)
        m = jnp.min(d2, axis=1, keepdims=True)
        out = jnp.where(ik == u + 1, am + b * N, out)
    idx_ref[...] = out


def _knn(coords_t, b):
    return pl.pallas_call(
        functools.partial(_knn_body, b),
        grid=(N // _BQ,),
        in_specs=[
            pl.BlockSpec((1, 3, _BQ), lambda i: (0, 0, i)),
            pl.BlockSpec((1, 3, N), lambda i: (0, 0, 0)),
        ],
        out_specs=pl.BlockSpec((_BQ, IDXW), lambda i: (i, 0)),
        out_shape=jax.ShapeDtypeStruct((N, IDXW), I32),
    )(coords_t[b:b + 1], coords_t[b:b + 1])


_NW = 32
_RW = N // _NW
_L = 16
_NSLOT = 4


def _sc_gather_body(idx_hbm, t_hbm, g_hbm, s1_hbm, q1_hbm,
                    idxall_v, rows_v, stat_v, gsem, osem):
    wid = lax.axis_index("s") * 2 + lax.axis_index("c")
    base = wid * _RW
    nacc = C // _L
    pltpu.sync_copy(idx_hbm.at[pl.ds(base * IDXW, _RW * IDXW)], idxall_v)

    def ids(r):
        return idxall_v.at[pl.ds(r * IDXW, GL)]

    def gcopy(r, s):
        return pltpu.make_async_copy(t_hbm.at[ids(r)], rows_v.at[s], gsem)

    def ocopy(r, s):
        return pltpu.make_async_copy(
            rows_v.at[s, pl.ds(1, KN)],
            g_hbm.at[pl.ds((base + r) * KN, KN)], osem)

    gcopy(0, 0).start()
    gcopy(1, 1).start()

    def step(r, s, acc):
        gcopy(r, s).wait()
        acc = list(acc)
        bm = [rows_v[s, 0, pl.ds(c * _L, _L)] for c in range(nacc)]
        for j in range(1, GL):
            for c in range(nacc):
                v = rows_v[s, j, pl.ds(c * _L, _L)] + bm[c]
                rows_v[s, j, pl.ds(c * _L, _L)] = v
                acc[c] = acc[c] + v
                acc[nacc + c] = acc[nacc + c] + v * v
        ocopy(r, s).start()

        @pl.when(r >= 2)
        def _():
            ocopy(r - 2, (s + 2) % _NSLOT).wait()

        @pl.when(r + 2 < _RW)
        def _():
            gcopy(r + 2, (s + 2) % _NSLOT).start()

        return tuple(acc)

    zeros = tuple(jnp.zeros((_L,), F32) for _ in range(2 * nacc))

    def quad(q, acc):
        r = q * _NSLOT
        for s in range(_NSLOT):
            acc = step(r + s, s, acc)
        return acc

    acc = lax.fori_loop(0, _RW // _NSLOT, quad, zeros)
    ocopy(_RW - 2, (_RW - 2) % _NSLOT).wait()
    ocopy(_RW - 1, (_RW - 1) % _NSLOT).wait()
    for c in range(nacc):
        stat_v[pl.ds(c * _L, _L)] = acc[c]
    pltpu.sync_copy(stat_v, s1_hbm.at[pl.ds(wid * C, C)])
    for c in range(nacc):
        stat_v[pl.ds(c * _L, _L)] = acc[nacc + c]
    pltpu.sync_copy(stat_v, q1_hbm.at[pl.ds(wid * C, C)])


def _sc_gather(idx_b, T):
    mesh = plsc.VectorSubcoreMesh(core_axis_name="c", subcore_axis_name="s")
    call = pl.kernel(
        _sc_gather_body,
        out_type=(
            jax.ShapeDtypeStruct((N * KN, C), F32),
            jax.ShapeDtypeStruct((_NW * C,), F32),
            jax.ShapeDtypeStruct((_NW * C,), F32),
        ),
        mesh=mesh,
        scratch_types=[
            pltpu.VMEM((_RW * IDXW,), I32),
            pltpu.VMEM((_NSLOT, GL, C), F32),
            pltpu.VMEM((C,), F32),
            pltpu.SemaphoreType.DMA,
            pltpu.SemaphoreType.DMA,
        ],
    )
    return call(idx_b.reshape(N * IDXW), T)


_DG = 64


def _dense_body(s1_ref, q1_ref, w2_ref, g1_ref, b1_ref, g_ref,
                mx_ref, mn_ref, sq2_ref, acc_ref):
    i = pl.program_id(0)
    inv = jnp.float32(1.0 / CNT)
    s1 = jnp.sum(s1_ref[...], axis=0) * inv
    q1 = jnp.sum(q1_ref[...], axis=0) * inv
    v1 = q1 - s1 * s1
    rs1 = lax.rsqrt(v1 + 1e-5) * g1_ref[0]
    sh1 = b1_ref[0] - s1 * rs1
    y1 = g_ref[...]
    h = jnp.maximum(y1 * rs1[None, :] + sh1[None, :], 0.0)
    y2 = lax.dot_general(h, w2_ref[...], (((1,), (1,)), ((), ())))
    yg = y2.reshape(_DG, KN, C)
    mx_ref[...] = jnp.max(yg, axis=1)
    mn_ref[...] = jnp.min(yg, axis=1)
    part = jnp.concatenate(
        [jnp.sum(y2, axis=0, keepdims=True),
         jnp.sum(y2 * y2, axis=0, keepdims=True)], axis=0)

    @pl.when(i == 0)
    def _():
        acc_ref[...] = part

    @pl.when(i > 0)
    def _():
        acc_ref[...] = acc_ref[...] + part

    @pl.when(i == pl.num_programs(0) - 1)
    def _():
        sq2_ref[...] = acc_ref[...]


def _dense(s1p, q1p, W2, g1, b1, G):
    return pl.pallas_call(
        _dense_body,
        grid=(N // _DG,),
        in_specs=[
            pl.BlockSpec((4 * _NW, C), lambda i: (0, 0)),
            pl.BlockSpec((4 * _NW, C), lambda i: (0, 0)),
            pl.BlockSpec((C, C), lambda i: (0, 0)),
            pl.BlockSpec((1, C), lambda i: (0, 0)),
            pl.BlockSpec((1, C), lambda i: (0, 0)),
            pl.BlockSpec((_DG * KN, C), lambda i: (i, 0)),
        ],
        out_specs=[
            pl.BlockSpec((_DG, C), lambda i: (i, 0)),
            pl.BlockSpec((_DG, C), lambda i: (i, 0)),
            pl.BlockSpec((2, C), lambda i: (0, 0)),
        ],
        out_shape=[
            jax.ShapeDtypeStruct((N, C), F32),
            jax.ShapeDtypeStruct((N, C), F32),
            jax.ShapeDtypeStruct((2, C), F32),
        ],
        scratch_shapes=[pltpu.VMEM((2, C), F32)],
    )(s1p, q1p, W2, g1, b1, G)


_FR = 512


def _final_body(mx_ref, mn_ref, sq2_ref, g2_ref, b2_ref, o_ref):
    inv = jnp.float32(1.0 / CNT)
    sq2 = sq2_ref[...]
    s2 = (sq2[0] + sq2[2] + sq2[4] + sq2[6]) * inv
    q2 = (sq2[1] + sq2[3] + sq2[5] + sq2[7]) * inv
    v2 = q2 - s2 * s2
    g2 = g2_ref[0]
    scale = lax.rsqrt(v2 + 1e-5) * g2
    shift = b2_ref[0] - s2 * scale
    sel = jnp.where((g2 >= 0.0)[None, :], mx_ref[...], mn_ref[...])
    o_ref[...] = jnp.maximum(sel * scale[None, :] + shift[None, :], 0.0)


def _final(Mx, Mn, sq2all, g2, b2):
    return pl.pallas_call(
        _final_body,
        grid=(N // _FR,),
        in_specs=[
            pl.BlockSpec((_FR, C), lambda i: (i, 0)),
            pl.BlockSpec((_FR, C), lambda i: (i, 0)),
            pl.BlockSpec((8, C), lambda i: (0, 0)),
            pl.BlockSpec((1, C), lambda i: (0, 0)),
            pl.BlockSpec((1, C), lambda i: (0, 0)),
        ],
        out_specs=pl.BlockSpec((_FR, C), lambda i: (i, 0)),
        out_shape=jax.ShapeDtypeStruct((N, C), F32),
    )(Mx, Mn, sq2all, g2, b2)


def kernel(x, coordinates, W1, g1, b1, W2, g2, b2):
    coords_t = jnp.transpose(coordinates, (0, 2, 1))
    A, Bm = _prep(x.reshape(BN, D), W1)
    T = jnp.concatenate([A, Bm], axis=0)
    g1r, b1r = g1.reshape(1, C), b1.reshape(1, C)
    g2r, b2r = g2.reshape(1, C), b2.reshape(1, C)

    idxs = [_knn(coords_t, b) for b in range(B)]
    sc = [_sc_gather(idxs[b], T) for b in range(B)]
    s1p = jnp.concatenate([s.reshape(_NW, C) for _, s, _ in sc], axis=0)
    q1p = jnp.concatenate([q.reshape(_NW, C) for _, _, q in sc], axis=0)

    dense = [_dense(s1p, q1p, W2, g1r, b1r, sc[b][0]) for b in range(B)]
    sq2all = jnp.concatenate([d[2] for d in dense], axis=0)
    ys = [_final(dense[b][0], dense[b][1], sq2all, g2r, b2r) for b in range(B)]
    y = jnp.stack(ys, axis=0)
    return (y, coordinates)


# speedup vs baseline: 1.1270x; 1.1270x over previous
import functools
import jax
import jax.numpy as jnp
from jax import lax
from jax.experimental import pallas as pl
from jax.experimental.pallas import tpu as pltpu
from jax.experimental.pallas import tpu_sc as plsc

B, N, D = 4, 2048, 64
C = 2 * D
KN = 32
BN = B * N
CNT = BN * KN
IDXW = 40
GL = KN + 1
F32 = jnp.float32
I32 = jnp.int32

_PREP_ROWS = 512


def _prep_body(x_ref, w1_ref, a_ref, bm_ref):
    x = x_ref[...]
    w1a = w1_ref[:, :D]
    w1d = w1_ref[:, D:] - w1a
    dn = (((1,), (1,)), ((), ()))
    a_ref[...] = lax.dot_general(x, w1a, dn)
    bm_ref[...] = lax.dot_general(x, w1d, dn)


def _prep(x_flat, W1):
    return pl.pallas_call(
        _prep_body,
        grid=(BN // _PREP_ROWS,),
        in_specs=[
            pl.BlockSpec((_PREP_ROWS, D), lambda i: (i, 0)),
            pl.BlockSpec((C, C), lambda i: (0, 0)),
        ],
        out_specs=[
            pl.BlockSpec((_PREP_ROWS, C), lambda i: (i, 0)),
            pl.BlockSpec((_PREP_ROWS, C), lambda i: (i, 0)),
        ],
        out_shape=[
            jax.ShapeDtypeStruct((BN, C), F32),
            jax.ShapeDtypeStruct((BN, C), F32),
        ],
    )(x_flat, W1)


_BQ = 512


def _knn_body(b, cq_ref, cf_ref, idx_ref):
    i = pl.program_id(0)
    cq = cq_ref[0]
    cf = cf_ref[0]
    sqq = jnp.sum(cq * cq, axis=0)[:, None]
    sqf = jnp.sum(cf * cf, axis=0)[None, :]
    dot = lax.dot_general(cq, cf, (((0,), (0,)), ((), ())))
    d2 = sqq + sqf - 2.0 * dot
    iota = lax.broadcasted_iota(I32, (_BQ, N), 1)
    ik = lax.broadcasted_iota(I32, (_BQ, IDXW), 1)
    inf = jnp.float32(jnp.inf)

    row = lax.broadcasted_iota(I32, (_BQ, IDXW), 0) + (i * _BQ + (B + b) * N)
    out = jnp.where(ik == 0, row, 0)
    m = jnp.min(d2, axis=1, keepdims=True)
    for u in range(KN):
        cand = jnp.where(d2 == m, iota, N)
        am = jnp.min(cand, axis=1, keepdims=True)
        d2 = jnp.where(iota == am, inf, d2)
        m = jnp.min(d2, axis=1, keepdims=True)
        out = jnp.where(ik == u + 1, am + b * N, out)
    idx_ref[...] = out


def _knn(coords_t, b):
    return pl.pallas_call(
        functools.partial(_knn_body, b),
        grid=(N // _BQ,),
        in_specs=[
            pl.BlockSpec((1, 3, _BQ), lambda i: (0, 0, i)),
            pl.BlockSpec((1, 3, N), lambda i: (0, 0, 0)),
        ],
        out_specs=pl.BlockSpec((_BQ, IDXW), lambda i: (i, 0)),
        out_shape=jax.ShapeDtypeStruct((N, IDXW), I32),
    )(coords_t[b:b + 1], coords_t[b:b + 1])


_NW = 32
_RW = N // _NW
_L = 16
_NSLOT = 4


def _sc_gather_body(idx_hbm, t_hbm, g_hbm, s1_hbm, q1_hbm,
                    idxall_v, rows_v, stat_v, gsem, osem):
    wid = lax.axis_index("s") * 2 + lax.axis_index("c")
    base = wid * _RW
    nacc = C // _L
    pltpu.sync_copy(idx_hbm.at[pl.ds(base * IDXW, _RW * IDXW)], idxall_v)

    def ids(r):
        return idxall_v.at[pl.ds(r * IDXW, GL)]

    def gcopy(r, s):
        return pltpu.make_async_copy(t_hbm.at[ids(r)], rows_v.at[s], gsem)

    def ocopy(r, s):
        return pltpu.make_async_copy(
            rows_v.at[s, pl.ds(1, KN)],
            g_hbm.at[pl.ds((base + r) * KN, KN)], osem)

    gcopy(0, 0).start()
    gcopy(1, 1).start()

    def step(r, s, acc):
        gcopy(r, s).wait()
        acc = list(acc)
        bm = [rows_v[s, 0, pl.ds(c * _L, _L)] for c in range(nacc)]
        for j in range(1, GL):
            for c in range(nacc):
                v = rows_v[s, j, pl.ds(c * _L, _L)] + bm[c]
                rows_v[s, j, pl.ds(c * _L, _L)] = v
                acc[c] = acc[c] + v
                acc[nacc + c] = acc[nacc + c] + v * v
        ocopy(r, s).start()

        @pl.when(r >= 2)
        def _():
            ocopy(r - 2, (s + 2) % _NSLOT).wait()

        @pl.when(r + 2 < _RW)
        def _():
            gcopy(r + 2, (s + 2) % _NSLOT).start()

        return tuple(acc)

    zeros = tuple(jnp.zeros((_L,), F32) for _ in range(2 * nacc))

    def quad(q, acc):
        r = q * _NSLOT
        for s in range(_NSLOT):
            acc = step(r + s, s, acc)
        return acc

    acc = lax.fori_loop(0, _RW // _NSLOT, quad, zeros)
    ocopy(_RW - 2, (_RW - 2) % _NSLOT).wait()
    ocopy(_RW - 1, (_RW - 1) % _NSLOT).wait()
    for c in range(nacc):
        stat_v[pl.ds(c * _L, _L)] = acc[c]
    pltpu.sync_copy(stat_v, s1_hbm.at[pl.ds(wid * C, C)])
    for c in range(nacc):
        stat_v[pl.ds(c * _L, _L)] = acc[nacc + c]
    pltpu.sync_copy(stat_v, q1_hbm.at[pl.ds(wid * C, C)])


def _sc_gather(idx_b, T):
    mesh = plsc.VectorSubcoreMesh(core_axis_name="c", subcore_axis_name="s")
    call = pl.kernel(
        _sc_gather_body,
        out_type=(
            jax.ShapeDtypeStruct((N * KN, C), F32),
            jax.ShapeDtypeStruct((_NW * C,), F32),
            jax.ShapeDtypeStruct((_NW * C,), F32),
        ),
        mesh=mesh,
        scratch_types=[
            pltpu.VMEM((_RW * IDXW,), I32),
            pltpu.VMEM((_NSLOT, GL, C), F32),
            pltpu.VMEM((C,), F32),
            pltpu.SemaphoreType.DMA,
            pltpu.SemaphoreType.DMA,
        ],
    )
    return call(idx_b.reshape(N * IDXW), T)


_DG = 128


def _dense_body(s1_ref, q1_ref, w2_ref, g1_ref, b1_ref, g_ref,
                mx_ref, mn_ref, sq2_ref, acc_ref):
    i = pl.program_id(0)
    inv = jnp.float32(1.0 / CNT)
    s1 = jnp.sum(s1_ref[...], axis=0) * inv
    q1 = jnp.sum(q1_ref[...], axis=0) * inv
    v1 = q1 - s1 * s1
    rs1 = lax.rsqrt(v1 + 1e-5) * g1_ref[0]
    sh1 = b1_ref[0] - s1 * rs1
    y1 = g_ref[...]
    h = jnp.maximum(y1 * rs1[None, :] + sh1[None, :], 0.0)
    y2 = lax.dot_general(h, w2_ref[...], (((1,), (1,)), ((), ())))
    yg = y2.reshape(_DG, KN, C)
    mx_ref[...] = jnp.max(yg, axis=1)
    mn_ref[...] = jnp.min(yg, axis=1)
    part = jnp.concatenate(
        [jnp.sum(y2, axis=0, keepdims=True),
         jnp.sum(y2 * y2, axis=0, keepdims=True)], axis=0)

    @pl.when(i == 0)
    def _():
        acc_ref[...] = part

    @pl.when(i > 0)
    def _():
        acc_ref[...] = acc_ref[...] + part

    @pl.when(i == pl.num_programs(0) - 1)
    def _():
        sq2_ref[...] = acc_ref[...]


def _dense(s1p, q1p, W2, g1, b1, G):
    return pl.pallas_call(
        _dense_body,
        grid=(N // _DG,),
        in_specs=[
            pl.BlockSpec((4 * _NW, C), lambda i: (0, 0)),
            pl.BlockSpec((4 * _NW, C), lambda i: (0, 0)),
            pl.BlockSpec((C, C), lambda i: (0, 0)),
            pl.BlockSpec((1, C), lambda i: (0, 0)),
            pl.BlockSpec((1, C), lambda i: (0, 0)),
            pl.BlockSpec((_DG * KN, C), lambda i: (i, 0)),
        ],
        out_specs=[
            pl.BlockSpec((_DG, C), lambda i: (i, 0)),
            pl.BlockSpec((_DG, C), lambda i: (i, 0)),
            pl.BlockSpec((2, C), lambda i: (0, 0)),
        ],
        out_shape=[
            jax.ShapeDtypeStruct((N, C), F32),
            jax.ShapeDtypeStruct((N, C), F32),
            jax.ShapeDtypeStruct((2, C), F32),
        ],
        scratch_shapes=[pltpu.VMEM((2, C), F32)],
    )(s1p, q1p, W2, g1, b1, G)


_FR = 512


def _final_body(mx_ref, mn_ref, sq2_ref, g2_ref, b2_ref, o_ref):
    inv = jnp.float32(1.0 / CNT)
    sq2 = sq2_ref[...]
    s2 = (sq2[0] + sq2[2] + sq2[4] + sq2[6]) * inv
    q2 = (sq2[1] + sq2[3] + sq2[5] + sq2[7]) * inv
    v2 = q2 - s2 * s2
    g2 = g2_ref[0]
    scale = lax.rsqrt(v2 + 1e-5) * g2
    shift = b2_ref[0] - s2 * scale
    sel = jnp.where((g2 >= 0.0)[None, :], mx_ref[...], mn_ref[...])
    o_ref[...] = jnp.maximum(sel * scale[None, :] + shift[None, :], 0.0)


def _final(Mx, Mn, sq2all, g2, b2):
    return pl.pallas_call(
        _final_body,
        grid=(N // _FR,),
        in_specs=[
            pl.BlockSpec((_FR, C), lambda i: (i, 0)),
            pl.BlockSpec((_FR, C), lambda i: (i, 0)),
            pl.BlockSpec((8, C), lambda i: (0, 0)),
            pl.BlockSpec((1, C), lambda i: (0, 0)),
            pl.BlockSpec((1, C), lambda i: (0, 0)),
        ],
        out_specs=pl.BlockSpec((_FR, C), lambda i: (i, 0)),
        out_shape=jax.ShapeDtypeStruct((N, C), F32),
    )(Mx, Mn, sq2all, g2, b2)


def kernel(x, coordinates, W1, g1, b1, W2, g2, b2):
    coords_t = jnp.transpose(coordinates, (0, 2, 1))
    A, Bm = _prep(x.reshape(BN, D), W1)
    T = jnp.concatenate([A, Bm], axis=0)
    g1r, b1r = g1.reshape(1, C), b1.reshape(1, C)
    g2r, b2r = g2.reshape(1, C), b2.reshape(1, C)

    idxs = [_knn(coords_t, b) for b in range(B)]
    sc = [_sc_gather(idxs[b], T) for b in range(B)]
    s1p = jnp.concatenate([s.reshape(_NW, C) for _, s, _ in sc], axis=0)
    q1p = jnp.concatenate([q.reshape(_NW, C) for _, _, q in sc], axis=0)

    dense = [_dense(s1p, q1p, W2, g1r, b1r, sc[b][0]) for b in range(B)]
    sq2all = jnp.concatenate([d[2] for d in dense], axis=0)
    ys = [_final(dense[b][0], dense[b][1], sq2all, g2r, b2r) for b in range(B)]
    y = jnp.stack(ys, axis=0)
    return (y, coordinates)


# speedup vs baseline: 1.1439x; 1.0150x over previous
import functools
import jax
import jax.numpy as jnp
from jax import lax
from jax.experimental import pallas as pl
from jax.experimental.pallas import tpu as pltpu
from jax.experimental.pallas import tpu_sc as plsc

B, N, D = 4, 2048, 64
C = 2 * D
KN = 32
BN = B * N
CNT = BN * KN
IDXW = 40
GL = KN + 1
F32 = jnp.float32
I32 = jnp.int32

_PREP_ROWS = 512


def _prep_body(x_ref, w1_ref, a_ref, bm_ref):
    x = x_ref[...]
    w1a = w1_ref[:, :D]
    w1d = w1_ref[:, D:] - w1a
    dn = (((1,), (1,)), ((), ()))
    a_ref[...] = lax.dot_general(x, w1a, dn)
    bm_ref[...] = lax.dot_general(x, w1d, dn)


def _prep(x_flat, W1):
    return pl.pallas_call(
        _prep_body,
        grid=(BN // _PREP_ROWS,),
        in_specs=[
            pl.BlockSpec((_PREP_ROWS, D), lambda i: (i, 0)),
            pl.BlockSpec((C, C), lambda i: (0, 0)),
        ],
        out_specs=[
            pl.BlockSpec((_PREP_ROWS, C), lambda i: (i, 0)),
            pl.BlockSpec((_PREP_ROWS, C), lambda i: (i, 0)),
        ],
        out_shape=[
            jax.ShapeDtypeStruct((BN, C), F32),
            jax.ShapeDtypeStruct((BN, C), F32),
        ],
    )(x_flat, W1)


_BQ = 512


def _knn_body(b, cq_ref, cf_ref, idx_ref):
    i = pl.program_id(0)
    cq = cq_ref[0]
    cf = cf_ref[0]
    sqq = jnp.sum(cq * cq, axis=0)[:, None]
    sqf = jnp.sum(cf * cf, axis=0)[None, :]
    dot = lax.dot_general(cq, cf, (((0,), (0,)), ((), ())))
    d2 = sqq + sqf - 2.0 * dot
    iota = lax.broadcasted_iota(I32, (_BQ, N), 1)
    ik = lax.broadcasted_iota(I32, (_BQ, IDXW), 1)
    inf = jnp.float32(jnp.inf)

    row = lax.broadcasted_iota(I32, (_BQ, IDXW), 0) + (i * _BQ + (B + b) * N)
    out = jnp.where(ik == 0, row, 0)
    m = jnp.min(d2, axis=1, keepdims=True)
    for u in range(KN):
        cand = jnp.where(d2 == m, iota, N)
        am = jnp.min(cand, axis=1, keepdims=True)
        d2 = jnp.where(iota == am, inf, d2)
        m = jnp.min(d2, axis=1, keepdims=True)
        out = jnp.where(ik == u + 1, am + b * N, out)
    idx_ref[...] = out


def _knn(coords_t, b):
    return pl.pallas_call(
        functools.partial(_knn_body, b),
        grid=(N // _BQ,),
        in_specs=[
            pl.BlockSpec((1, 3, _BQ), lambda i: (0, 0, i)),
            pl.BlockSpec((1, 3, N), lambda i: (0, 0, 0)),
        ],
        out_specs=pl.BlockSpec((_BQ, IDXW), lambda i: (i, 0)),
        out_shape=jax.ShapeDtypeStruct((N, IDXW), I32),
    )(coords_t[b:b + 1], coords_t[b:b + 1])


_NW = 32
_RW = N // _NW
_L = 16
_NSLOT = 4


def _sc_gather_body(idx_hbm, t_hbm, g_hbm, s1_hbm, q1_hbm,
                    idxall_v, rows_v, stat_v, gsem, osem):
    wid = lax.axis_index("s") * 2 + lax.axis_index("c")
    base = wid * _RW
    nacc = C // _L
    pltpu.sync_copy(idx_hbm.at[pl.ds(base * IDXW, _RW * IDXW)], idxall_v)

    def ids(r):
        return idxall_v.at[pl.ds(r * IDXW, GL)]

    def gcopy(r, s):
        return pltpu.make_async_copy(t_hbm.at[ids(r)], rows_v.at[s], gsem)

    def ocopy(r, s):
        return pltpu.make_async_copy(
            rows_v.at[s, pl.ds(1, KN)],
            g_hbm.at[pl.ds((base + r) * KN, KN)], osem)

    gcopy(0, 0).start()
    gcopy(1, 1).start()

    def step(r, s, acc):
        gcopy(r, s).wait()
        acc = list(acc)
        bm = [rows_v[s, 0, pl.ds(c * _L, _L)] for c in range(nacc)]
        for j in range(1, GL):
            for c in range(nacc):
                v = rows_v[s, j, pl.ds(c * _L, _L)] + bm[c]
                rows_v[s, j, pl.ds(c * _L, _L)] = v
                acc[c] = acc[c] + v
                acc[nacc + c] = acc[nacc + c] + v * v
        ocopy(r, s).start()

        @pl.when(r >= 2)
        def _():
            ocopy(r - 2, (s + 2) % _NSLOT).wait()

        @pl.when(r + 2 < _RW)
        def _():
            gcopy(r + 2, (s + 2) % _NSLOT).start()

        return tuple(acc)

    zeros = tuple(jnp.zeros((_L,), F32) for _ in range(2 * nacc))

    def quad(q, acc):
        r = q * _NSLOT
        for s in range(_NSLOT):
            acc = step(r + s, s, acc)
        return acc

    acc = lax.fori_loop(0, _RW // _NSLOT, quad, zeros)
    ocopy(_RW - 2, (_RW - 2) % _NSLOT).wait()
    ocopy(_RW - 1, (_RW - 1) % _NSLOT).wait()
    for c in range(nacc):
        stat_v[pl.ds(c * _L, _L)] = acc[c]
    pltpu.sync_copy(stat_v, s1_hbm.at[pl.ds(wid * C, C)])
    for c in range(nacc):
        stat_v[pl.ds(c * _L, _L)] = acc[nacc + c]
    pltpu.sync_copy(stat_v, q1_hbm.at[pl.ds(wid * C, C)])


def _sc_gather(idx_b, T):
    mesh = plsc.VectorSubcoreMesh(core_axis_name="c", subcore_axis_name="s")
    call = pl.kernel(
        _sc_gather_body,
        out_type=(
            jax.ShapeDtypeStruct((N * KN, C), F32),
            jax.ShapeDtypeStruct((_NW * C,), F32),
            jax.ShapeDtypeStruct((_NW * C,), F32),
        ),
        mesh=mesh,
        scratch_types=[
            pltpu.VMEM((_RW * IDXW,), I32),
            pltpu.VMEM((_NSLOT, GL, C), F32),
            pltpu.VMEM((C,), F32),
            pltpu.SemaphoreType.DMA,
            pltpu.SemaphoreType.DMA,
        ],
    )
    return call(idx_b.reshape(N * IDXW), T)


_DG = 256


def _dense_body(s1_ref, q1_ref, w2_ref, g1_ref, b1_ref, g_ref,
                mx_ref, mn_ref, sq2_ref, acc_ref):
    i = pl.program_id(0)
    inv = jnp.float32(1.0 / CNT)
    s1 = jnp.sum(s1_ref[...], axis=0) * inv
    q1 = jnp.sum(q1_ref[...], axis=0) * inv
    v1 = q1 - s1 * s1
    rs1 = lax.rsqrt(v1 + 1e-5) * g1_ref[0]
    sh1 = b1_ref[0] - s1 * rs1
    y1 = g_ref[...]
    h = jnp.maximum(y1 * rs1[None, :] + sh1[None, :], 0.0)
    y2 = lax.dot_general(h, w2_ref[...], (((1,), (1,)), ((), ())))
    yg = y2.reshape(_DG, KN, C)
    mx_ref[...] = jnp.max(yg, axis=1)
    mn_ref[...] = jnp.min(yg, axis=1)
    part = jnp.concatenate(
        [jnp.sum(y2, axis=0, keepdims=True),
         jnp.sum(y2 * y2, axis=0, keepdims=True)], axis=0)

    @pl.when(i == 0)
    def _():
        acc_ref[...] = part

    @pl.when(i > 0)
    def _():
        acc_ref[...] = acc_ref[...] + part

    @pl.when(i == pl.num_programs(0) - 1)
    def _():
        sq2_ref[...] = acc_ref[...]


def _dense(s1p, q1p, W2, g1, b1, G):
    return pl.pallas_call(
        _dense_body,
        grid=(N // _DG,),
        in_specs=[
            pl.BlockSpec((4 * _NW, C), lambda i: (0, 0)),
            pl.BlockSpec((4 * _NW, C), lambda i: (0, 0)),
            pl.BlockSpec((C, C), lambda i: (0, 0)),
            pl.BlockSpec((1, C), lambda i: (0, 0)),
            pl.BlockSpec((1, C), lambda i: (0, 0)),
            pl.BlockSpec((_DG * KN, C), lambda i: (i, 0)),
        ],
        out_specs=[
            pl.BlockSpec((_DG, C), lambda i: (i, 0)),
            pl.BlockSpec((_DG, C), lambda i: (i, 0)),
            pl.BlockSpec((2, C), lambda i: (0, 0)),
        ],
        out_shape=[
            jax.ShapeDtypeStruct((N, C), F32),
            jax.ShapeDtypeStruct((N, C), F32),
            jax.ShapeDtypeStruct((2, C), F32),
        ],
        scratch_shapes=[pltpu.VMEM((2, C), F32)],
    )(s1p, q1p, W2, g1, b1, G)


_FR = 512


def _final_body(mx_ref, mn_ref, sq2_ref, g2_ref, b2_ref, o_ref):
    inv = jnp.float32(1.0 / CNT)
    sq2 = sq2_ref[...]
    s2 = (sq2[0] + sq2[2] + sq2[4] + sq2[6]) * inv
    q2 = (sq2[1] + sq2[3] + sq2[5] + sq2[7]) * inv
    v2 = q2 - s2 * s2
    g2 = g2_ref[0]
    scale = lax.rsqrt(v2 + 1e-5) * g2
    shift = b2_ref[0] - s2 * scale
    sel = jnp.where((g2 >= 0.0)[None, :], mx_ref[...], mn_ref[...])
    o_ref[...] = jnp.maximum(sel * scale[None, :] + shift[None, :], 0.0)


def _final(Mx, Mn, sq2all, g2, b2):
    return pl.pallas_call(
        _final_body,
        grid=(N // _FR,),
        in_specs=[
            pl.BlockSpec((_FR, C), lambda i: (i, 0)),
            pl.BlockSpec((_FR, C), lambda i: (i, 0)),
            pl.BlockSpec((8, C), lambda i: (0, 0)),
            pl.BlockSpec((1, C), lambda i: (0, 0)),
            pl.BlockSpec((1, C), lambda i: (0, 0)),
        ],
        out_specs=pl.BlockSpec((_FR, C), lambda i: (i, 0)),
        out_shape=jax.ShapeDtypeStruct((N, C), F32),
    )(Mx, Mn, sq2all, g2, b2)


def kernel(x, coordinates, W1, g1, b1, W2, g2, b2):
    coords_t = jnp.transpose(coordinates, (0, 2, 1))
    A, Bm = _prep(x.reshape(BN, D), W1)
    T = jnp.concatenate([A, Bm], axis=0)
    g1r, b1r = g1.reshape(1, C), b1.reshape(1, C)
    g2r, b2r = g2.reshape(1, C), b2.reshape(1, C)

    idxs = [_knn(coords_t, b) for b in range(B)]
    sc = [_sc_gather(idxs[b], T) for b in range(B)]
    s1p = jnp.concatenate([s.reshape(_NW, C) for _, s, _ in sc], axis=0)
    q1p = jnp.concatenate([q.reshape(_NW, C) for _, _, q in sc], axis=0)

    dense = [_dense(s1p, q1p, W2, g1r, b1r, sc[b][0]) for b in range(B)]
    sq2all = jnp.concatenate([d[2] for d in dense], axis=0)
    ys = [_final(dense[b][0], dense[b][1], sq2all, g2r, b2r) for b in range(B)]
    y = jnp.stack(ys, axis=0)
    return (y, coordinates)


# speedup vs baseline: 1.1492x; 1.0047x over previous
import functools
import jax
import jax.numpy as jnp
from jax import lax
from jax.experimental import pallas as pl
from jax.experimental.pallas import tpu as pltpu
from jax.experimental.pallas import tpu_sc as plsc

B, N, D = 4, 2048, 64
C = 2 * D
KN = 32
BN = B * N
CNT = BN * KN
IDXW = 40
GL = KN + 1
F32 = jnp.float32
I32 = jnp.int32

_PREP_ROWS = 512


def _prep_body(x_ref, w1_ref, a_ref, bm_ref):
    x = x_ref[...]
    w1a = w1_ref[:, :D]
    w1d = w1_ref[:, D:] - w1a
    dn = (((1,), (1,)), ((), ()))
    a_ref[...] = lax.dot_general(x, w1a, dn)
    bm_ref[...] = lax.dot_general(x, w1d, dn)


def _prep(x_flat, W1):
    return pl.pallas_call(
        _prep_body,
        grid=(BN // _PREP_ROWS,),
        in_specs=[
            pl.BlockSpec((_PREP_ROWS, D), lambda i: (i, 0)),
            pl.BlockSpec((C, C), lambda i: (0, 0)),
        ],
        out_specs=[
            pl.BlockSpec((_PREP_ROWS, C), lambda i: (i, 0)),
            pl.BlockSpec((_PREP_ROWS, C), lambda i: (i, 0)),
        ],
        out_shape=[
            jax.ShapeDtypeStruct((BN, C), F32),
            jax.ShapeDtypeStruct((BN, C), F32),
        ],
    )(x_flat, W1)


_BQ = 512


def _knn_body(b, cq_ref, cf_ref, idx_ref):
    i = pl.program_id(0)
    cq = cq_ref[0]
    cf = cf_ref[0]
    sqq = jnp.sum(cq * cq, axis=0)[:, None]
    sqf = jnp.sum(cf * cf, axis=0)[None, :]
    dot = lax.dot_general(cq, cf, (((0,), (0,)), ((), ())))
    d2 = sqq + sqf - 2.0 * dot
    iota = lax.broadcasted_iota(I32, (_BQ, N), 1)
    ik = lax.broadcasted_iota(I32, (_BQ, IDXW), 1)
    inf = jnp.float32(jnp.inf)

    row = lax.broadcasted_iota(I32, (_BQ, IDXW), 0) + (i * _BQ + (B + b) * N)
    out = jnp.where(ik == 0, row, 0)
    m = jnp.min(d2, axis=1, keepdims=True)
    for u in range(KN):
        cand = jnp.where(d2 == m, iota, N)
        am = jnp.min(cand, axis=1, keepdims=True)
        d2 = jnp.where(iota == am, inf, d2)
        m = jnp.min(d2, axis=1, keepdims=True)
        out = jnp.where(ik == u + 1, am + b * N, out)
    idx_ref[...] = out


def _knn(coords_t, b):
    return pl.pallas_call(
        functools.partial(_knn_body, b),
        grid=(N // _BQ,),
        in_specs=[
            pl.BlockSpec((1, 3, _BQ), lambda i: (0, 0, i)),
            pl.BlockSpec((1, 3, N), lambda i: (0, 0, 0)),
        ],
        out_specs=pl.BlockSpec((_BQ, IDXW), lambda i: (i, 0)),
        out_shape=jax.ShapeDtypeStruct((N, IDXW), I32),
    )(coords_t[b:b + 1], coords_t[b:b + 1])


_NW = 32
_RW = N // _NW
_L = 16
_NSLOT = 4


def _sc_gather_body(idx_hbm, t_hbm, g_hbm, s1_hbm, q1_hbm,
                    idxall_v, rows_v, stat_v, gsem, osem):
    wid = lax.axis_index("s") * 2 + lax.axis_index("c")
    base = wid * _RW
    nacc = C // _L
    pltpu.sync_copy(idx_hbm.at[pl.ds(base * IDXW, _RW * IDXW)], idxall_v)

    def ids(r):
        return idxall_v.at[pl.ds(r * IDXW, GL)]

    def gcopy(r, s):
        return pltpu.make_async_copy(t_hbm.at[ids(r)], rows_v.at[s], gsem)

    def ocopy(r, s):
        return pltpu.make_async_copy(
            rows_v.at[s, pl.ds(1, KN)],
            g_hbm.at[pl.ds((base + r) * KN, KN)], osem)

    gcopy(0, 0).start()
    gcopy(1, 1).start()

    def step(r, s, acc):
        gcopy(r, s).wait()
        acc = list(acc)
        bm = [rows_v[s, 0, pl.ds(c * _L, _L)] for c in range(nacc)]
        for j in range(1, GL):
            for c in range(nacc):
                v = rows_v[s, j, pl.ds(c * _L, _L)] + bm[c]
                rows_v[s, j, pl.ds(c * _L, _L)] = v
                acc[c] = acc[c] + v
                acc[nacc + c] = acc[nacc + c] + v * v
        ocopy(r, s).start()

        @pl.when(r >= 2)
        def _():
            ocopy(r - 2, (s + 2) % _NSLOT).wait()

        @pl.when(r + 2 < _RW)
        def _():
            gcopy(r + 2, (s + 2) % _NSLOT).start()

        return tuple(acc)

    zeros = tuple(jnp.zeros((_L,), F32) for _ in range(2 * nacc))

    def quad(q, acc):
        r = q * _NSLOT
        for s in range(_NSLOT):
            acc = step(r + s, s, acc)
        return acc

    acc = lax.fori_loop(0, _RW // _NSLOT, quad, zeros)
    ocopy(_RW - 2, (_RW - 2) % _NSLOT).wait()
    ocopy(_RW - 1, (_RW - 1) % _NSLOT).wait()
    for c in range(nacc):
        stat_v[pl.ds(c * _L, _L)] = acc[c]
    pltpu.sync_copy(stat_v, s1_hbm.at[pl.ds(wid * C, C)])
    for c in range(nacc):
        stat_v[pl.ds(c * _L, _L)] = acc[nacc + c]
    pltpu.sync_copy(stat_v, q1_hbm.at[pl.ds(wid * C, C)])


def _sc_gather(idx_b, T):
    mesh = plsc.VectorSubcoreMesh(core_axis_name="c", subcore_axis_name="s")
    call = pl.kernel(
        _sc_gather_body,
        out_type=(
            jax.ShapeDtypeStruct((N * KN, C), F32),
            jax.ShapeDtypeStruct((_NW * C,), F32),
            jax.ShapeDtypeStruct((_NW * C,), F32),
        ),
        mesh=mesh,
        scratch_types=[
            pltpu.VMEM((_RW * IDXW,), I32),
            pltpu.VMEM((_NSLOT, GL, C), F32),
            pltpu.VMEM((C,), F32),
            pltpu.SemaphoreType.DMA,
            pltpu.SemaphoreType.DMA,
        ],
    )
    return call(idx_b.reshape(N * IDXW), T)


_DG = 512


def _dense_body(s1_ref, q1_ref, w2_ref, g1_ref, b1_ref, g_ref,
                mx_ref, mn_ref, sq2_ref, acc_ref):
    i = pl.program_id(0)
    inv = jnp.float32(1.0 / CNT)
    s1 = jnp.sum(s1_ref[...], axis=0) * inv
    q1 = jnp.sum(q1_ref[...], axis=0) * inv
    v1 = q1 - s1 * s1
    rs1 = lax.rsqrt(v1 + 1e-5) * g1_ref[0]
    sh1 = b1_ref[0] - s1 * rs1
    y1 = g_ref[...]
    h = jnp.maximum(y1 * rs1[None, :] + sh1[None, :], 0.0)
    y2 = lax.dot_general(h, w2_ref[...], (((1,), (1,)), ((), ())))
    yg = y2.reshape(_DG, KN, C)
    mx_ref[...] = jnp.max(yg, axis=1)
    mn_ref[...] = jnp.min(yg, axis=1)
    part = jnp.concatenate(
        [jnp.sum(y2, axis=0, keepdims=True),
         jnp.sum(y2 * y2, axis=0, keepdims=True)], axis=0)

    @pl.when(i == 0)
    def _():
        acc_ref[...] = part

    @pl.when(i > 0)
    def _():
        acc_ref[...] = acc_ref[...] + part

    @pl.when(i == pl.num_programs(0) - 1)
    def _():
        sq2_ref[...] = acc_ref[...]


def _dense(s1p, q1p, W2, g1, b1, G):
    return pl.pallas_call(
        _dense_body,
        grid=(N // _DG,),
        in_specs=[
            pl.BlockSpec((4 * _NW, C), lambda i: (0, 0)),
            pl.BlockSpec((4 * _NW, C), lambda i: (0, 0)),
            pl.BlockSpec((C, C), lambda i: (0, 0)),
            pl.BlockSpec((1, C), lambda i: (0, 0)),
            pl.BlockSpec((1, C), lambda i: (0, 0)),
            pl.BlockSpec((_DG * KN, C), lambda i: (i, 0)),
        ],
        out_specs=[
            pl.BlockSpec((_DG, C), lambda i: (i, 0)),
            pl.BlockSpec((_DG, C), lambda i: (i, 0)),
            pl.BlockSpec((2, C), lambda i: (0, 0)),
        ],
        out_shape=[
            jax.ShapeDtypeStruct((N, C), F32),
            jax.ShapeDtypeStruct((N, C), F32),
            jax.ShapeDtypeStruct((2, C), F32),
        ],
        scratch_shapes=[pltpu.VMEM((2, C), F32)],
    )(s1p, q1p, W2, g1, b1, G)


_FR = 512


def _final_body(mx_ref, mn_ref, sq2_ref, g2_ref, b2_ref, o_ref):
    inv = jnp.float32(1.0 / CNT)
    sq2 = sq2_ref[...]
    s2 = (sq2[0] + sq2[2] + sq2[4] + sq2[6]) * inv
    q2 = (sq2[1] + sq2[3] + sq2[5] + sq2[7]) * inv
    v2 = q2 - s2 * s2
    g2 = g2_ref[0]
    scale = lax.rsqrt(v2 + 1e-5) * g2
    shift = b2_ref[0] - s2 * scale
    sel = jnp.where((g2 >= 0.0)[None, :], mx_ref[...], mn_ref[...])
    o_ref[...] = jnp.maximum(sel * scale[None, :] + shift[None, :], 0.0)


def _final(Mx, Mn, sq2all, g2, b2):
    return pl.pallas_call(
        _final_body,
        grid=(N // _FR,),
        in_specs=[
            pl.BlockSpec((_FR, C), lambda i: (i, 0)),
            pl.BlockSpec((_FR, C), lambda i: (i, 0)),
            pl.BlockSpec((8, C), lambda i: (0, 0)),
            pl.BlockSpec((1, C), lambda i: (0, 0)),
            pl.BlockSpec((1, C), lambda i: (0, 0)),
        ],
        out_specs=pl.BlockSpec((_FR, C), lambda i: (i, 0)),
        out_shape=jax.ShapeDtypeStruct((N, C), F32),
    )(Mx, Mn, sq2all, g2, b2)


def kernel(x, coordinates, W1, g1, b1, W2, g2, b2):
    coords_t = jnp.transpose(coordinates, (0, 2, 1))
    A, Bm = _prep(x.reshape(BN, D), W1)
    T = jnp.concatenate([A, Bm], axis=0)
    g1r, b1r = g1.reshape(1, C), b1.reshape(1, C)
    g2r, b2r = g2.reshape(1, C), b2.reshape(1, C)

    idxs = [_knn(coords_t, b) for b in range(B)]
    sc = [_sc_gather(idxs[b], T) for b in range(B)]
    s1p = jnp.concatenate([s.reshape(_NW, C) for _, s, _ in sc], axis=0)
    q1p = jnp.concatenate([q.reshape(_NW, C) for _, _, q in sc], axis=0)

    dense = [_dense(s1p, q1p, W2, g1r, b1r, sc[b][0]) for b in range(B)]
    sq2all = jnp.concatenate([d[2] for d in dense], axis=0)
    ys = [_final(dense[b][0], dense[b][1], sq2all, g2r, b2r) for b in range(B)]
    y = jnp.stack(ys, axis=0)
    return (y, coordinates)
